# baseline (speedup 1.0000x reference)
import functools

import jax
import jax.numpy as jnp
from jax import lax
from jax.experimental import pallas as pl
from jax.experimental.pallas import tpu as pltpu
from jax.experimental.pallas import tpu_sc as plsc

NWORDS = 1000000
EMB = 64
BATCH = 16384

NC = 2
NS = 16
L = 16
NW = NC * NS

NBLK = (NWORDS + 127) // 128
NB = (NBLK + NW - 1) // NW
TAIL_BLK = NWORDS // 128
TAIL_START = TAIL_BLK * 128
GOUT = BATCH + 8
DUMP = GOUT - 1
FLUSH = 256
NVREG = BATCH // L


def _neg_log_sigmoid(s):
    z = -s
    m = jnp.maximum(z, 0.0)
    u = jnp.exp(-jnp.abs(z))
    t = u / (2.0 + u)
    t2 = t * t
    p = 1.0 + t2 * (1.0 / 3.0 + t2 * (1.0 / 5.0 + t2 * (1.0 / 7.0 + t2 * (1.0 / 9.0))))
    return m + 2.0 * t * p


def _gather_body(wt_hbm, tail_hbm, idx_hbm, g_hbm,
                 idx_v, ml_v, sub_v, cnt_v, slab_v, rows_v, mb_v, sem):
    wid = lax.axis_index("s") * NC + lax.axis_index("c")
    lo = wid * NB
    lanes = lax.iota(jnp.int32, L)

    pltpu.sync_copy(idx_hbm, idx_v.at[pl.ds(0, BATCH)])
    for q in range((NB + L - 1) // L):
        cnt_v[pl.ds(q * L, L)] = jnp.zeros((L,), jnp.int32)
    for q in range(FLUSH // L):
        mb_v[pl.ds(q * L, L)] = jnp.full((L,), DUMP, jnp.int32)

    def scan(v, n):
        iv = idx_v[pl.ds(v * L, L)]
        bv = (iv >> 7) - lo
        m = (bv >= 0) & (bv < NB)
        plsc.addupdate_scatter(cnt_v, [jnp.where(m, bv, 0)],
                               jnp.ones((L,), jnp.int32), mask=m)
        plsc.store_compressed(ml_v.at[pl.ds(n, L)], v * L + lanes, mask=m)
        pc = plsc.all_reduce_population_count(m)
        return n + pc[0]

    n = lax.fori_loop(0, NVREG, scan, 0)
    nv = (n + L - 1) // L

    def block(j, k):
        blk = lo + j
        c = cnt_v[pl.ds(j, L)][0]

        def do_block(k):
            def fetch_tail(_):
                pltpu.sync_copy(tail_hbm, slab_v)
                return 0

            def fetch_slab(_):
                pltpu.sync_copy(wt_hbm.at[:, pl.ds(blk * 128, 128)], slab_v)
                return 0

            lax.cond(blk == TAIL_BLK, fetch_tail, fetch_slab, 0)
            col_base = jnp.where(blk == TAIL_BLK, TAIL_START, blk * 128)

            def rescan(v, sn):
                mlv = plsc.load_gather(ml_v, [(v * L + lanes) & (BATCH - 1)])
                mlv = mlv & (BATCH - 1)
                ig = plsc.load_gather(idx_v, [mlv])
                mm = ((ig >> 7) == blk) & ((v * L + lanes) < n)
                plsc.store_compressed(sub_v.at[pl.ds(sn, L)], mlv, mask=mm)
                pc = plsc.all_reduce_population_count(mm)
                return sn + pc[0]

            sn = lax.fori_loop(0, nv, rescan, 0)

            def per_match(m2, k2):
                b = sub_v[pl.ds(m2, L)][0]
                r = idx_v[pl.ds(b, L)][0]
                col = r - col_base
                kk = k2 & (FLUSH - 1)
                for q in range(EMB // L):
                    vals = plsc.load_gather(
                        slab_v, [q * L + lanes, jnp.broadcast_to(col, (L,))])
                    rows_v[kk, pl.ds(q * L, L)] = vals
                plsc.store_scatter(mb_v, [jnp.broadcast_to(kk, (L,))],
                                   jnp.broadcast_to(b, (L,)),
                                   mask=lanes == 0)
                k2 = k2 + 1

                def flush(_):
                    pltpu.async_copy(rows_v, g_hbm.at[mb_v], sem).wait()
                    for q2 in range(FLUSH // L):
                        mb_v[pl.ds(q2 * L, L)] = jnp.full((L,), DUMP,
                                                          jnp.int32)
                    return 0

                lax.cond((k2 & (FLUSH - 1)) == 0, flush, lambda _: 0, 0)
                return k2

            return lax.fori_loop(0, sn, per_match, k)

        return lax.cond(c > 0, do_block, lambda kk: kk, k)

    lax.fori_loop(0, NB, block, 0)
    pltpu.async_copy(rows_v, g_hbm.at[mb_v], sem).wait()


def _combine_body(gw_hbm, gc_hbm, out_hbm, rw_v, rc_v, out_v, sem_w, sem_c):
    wid = lax.axis_index("s") * NC + lax.axis_index("c")
    lanes = lax.iota(jnp.int32, L)
    bpw = BATCH // NW
    ch = FLUSH

    for h in range(bpw // ch):
        base = wid * bpw + h * ch
        cp_w = pltpu.async_copy(gw_hbm.at[pl.ds(base, ch)], rw_v, sem_w)
        cp_c = pltpu.async_copy(gc_hbm.at[pl.ds(base, ch)], rc_v, sem_c)
        cp_w.wait()
        cp_c.wait()

        def group(g, carry):
            row = g * L + lanes
            acc = jnp.zeros((L,), jnp.float32)
            for e in range(EMB):
                col = jnp.full((L,), e, jnp.int32)
                w = plsc.load_gather(rw_v, [row, col])
                c = plsc.load_gather(rc_v, [row, col])
                acc = acc + w * c
            out_v[pl.ds(g * L, L)] = _neg_log_sigmoid(acc)
            return carry

        lax.fori_loop(0, ch // L, group, 0)
        pltpu.sync_copy(out_v, out_hbm.at[pl.ds(base, ch)])


_SC_PARAMS = pltpu.CompilerParams(needs_layout_passes=False,
                                  use_tc_tiling_on_sc=True)

_gather_call = functools.partial(
    pl.kernel,
    mesh=plsc.VectorSubcoreMesh(core_axis_name="c", subcore_axis_name="s"),
    out_type=jax.ShapeDtypeStruct((GOUT, 128), jnp.float32),
    scratch_types=[
        pltpu.VMEM((BATCH + L,), jnp.int32),
        pltpu.VMEM((BATCH + L,), jnp.int32),
        pltpu.VMEM((BATCH + L,), jnp.int32),
        pltpu.VMEM((NB + L,), jnp.int32),
        pltpu.VMEM((EMB, 128), jnp.float32),
        pltpu.VMEM((FLUSH, 128), jnp.float32),
        pltpu.VMEM((FLUSH,), jnp.int32),
        pltpu.SemaphoreType.DMA,
    ],
    compiler_params=_SC_PARAMS,
)(_gather_body)

_combine_call = functools.partial(
    pl.kernel,
    mesh=plsc.VectorSubcoreMesh(core_axis_name="c", subcore_axis_name="s"),
    out_type=jax.ShapeDtypeStruct((BATCH,), jnp.float32),
    scratch_types=[
        pltpu.VMEM((FLUSH, 128), jnp.float32),
        pltpu.VMEM((FLUSH, 128), jnp.float32),
        pltpu.VMEM((FLUSH,), jnp.float32),
        pltpu.SemaphoreType.DMA,
        pltpu.SemaphoreType.DMA,
    ],
    compiler_params=_SC_PARAMS,
)(_combine_body)


def _tail(W):
    return jnp.pad(W[TAIL_START:].T, ((0, 0), (0, 128 - (NWORDS - TAIL_START))))


def kernel(word_pos, context_pos, W_word, W_context):
    wp = word_pos.astype(jnp.int32)
    cp = context_pos.astype(jnp.int32)
    gw = _gather_call(W_word.T, _tail(W_word), wp)
    gc = _gather_call(W_context.T, _tail(W_context), cp)
    return _combine_call(gw, gc)


# speedup vs baseline: 1.6236x; 1.6236x over previous
import functools

import jax
import jax.numpy as jnp
from jax import lax
from jax.experimental import pallas as pl
from jax.experimental.pallas import tpu as pltpu
from jax.experimental.pallas import tpu_sc as plsc

NWORDS = 1000000
EMB = 64
BATCH = 16384

NC = 2
NS = 16
L = 16
NW = NC * NS

NBLK = (NWORDS + 127) // 128
NB = (NBLK + NW - 1) // NW
TAIL_BLK = NWORDS // 128
TAIL_START = TAIL_BLK * 128
GOUT = BATCH + 8
DUMP = GOUT - 1
FLUSH = 256
NVREG = BATCH // L


def _neg_log_sigmoid(s):
    z = -s
    m = jnp.maximum(z, 0.0)
    u = jnp.exp(-jnp.abs(z))
    t = u / (2.0 + u)
    t2 = t * t
    p = 1.0 + t2 * (1.0 / 3.0 + t2 * (1.0 / 5.0 + t2 * (1.0 / 7.0 + t2 * (1.0 / 9.0))))
    return m + 2.0 * t * p


RING = 4


def _gather_body(wt_hbm, tail_hbm, idx_hbm, g_hbm,
                 idx_v, ml_v, cnt_v, start_v, fill_v, blist_v, stage_v,
                 slab_v, rows_v, mb_v, sem0, sem1, sem2, sem3, sem_sc):
    sems = [sem0, sem1, sem2, sem3]
    wid = lax.axis_index("s") * NC + lax.axis_index("c")
    lo = wid * NB
    lanes = lax.iota(jnp.int32, L)

    pltpu.sync_copy(idx_hbm, idx_v.at[pl.ds(0, BATCH)])
    for q in range((NB + L - 1) // L):
        cnt_v[pl.ds(q * L, L)] = jnp.zeros((L,), jnp.int32)
    for q in range(FLUSH // L):
        mb_v[pl.ds(q * L, L)] = jnp.full((L,), DUMP, jnp.int32)

    def scan1(v, _):
        iv = idx_v[pl.ds(v * L, L)]
        bv = (iv >> 7) - lo
        m = (bv >= 0) & (bv < NB)
        plsc.addupdate_scatter(cnt_v, [jnp.where(m, bv, 0)],
                               jnp.ones((L,), jnp.int32), mask=m)
        return 0

    lax.fori_loop(0, NVREG, scan1, 0)

    carry = 0
    bn = 0
    for q in range((NB + L - 1) // L):
        cv = cnt_v[pl.ds(q * L, L)]
        cs = plsc.cumsum(cv)
        excl = carry + cs - cv
        start_v[pl.ds(q * L, L)] = excl
        fill_v[pl.ds(q * L, L)] = excl
        carry = carry + cs[L - 1]
        occ = (cv > 0) & (q * L + lanes < NB)
        plsc.store_compressed(blist_v.at[pl.ds(bn, L)], q * L + lanes,
                              mask=occ)
        pcq = plsc.all_reduce_population_count(occ)
        bn = bn + pcq[0]

    def scan2(v, _):
        iv = idx_v[pl.ds(v * L, L)]
        bv = (iv >> 7) - lo
        m = (bv >= 0) & (bv < NB)
        plsc.store_compressed(stage_v.at[pl.ds(0, L)], v * L + lanes,
                              mask=m)
        pc = plsc.all_reduce_population_count(m)

        def put(t, _2):
            b = stage_v[pl.ds(t, L)][0]
            jb = (idx_v[pl.ds(b, L)][0] >> 7) - lo
            p = fill_v[pl.ds(jb, L)][0]
            plsc.store_scatter(ml_v, [jnp.broadcast_to(p, (L,))],
                               jnp.broadcast_to(b, (L,)), mask=lanes == 0)
            plsc.store_scatter(fill_v, [jnp.broadcast_to(jb, (L,))],
                               jnp.broadcast_to(p + 1, (L,)),
                               mask=lanes == 0)
            return 0

        lax.fori_loop(0, pc[0], put, 0)
        return 0

    lax.fori_loop(0, NVREG, scan2, 0)

    def issue(i, s):
        jrel = blist_v[pl.ds(i, L)][0]
        blk = lo + jrel

        def fetch_tail(_):
            pltpu.async_copy(tail_hbm, slab_v.at[s], sems[s])
            return 0

        def fetch_slab(_):
            pltpu.async_copy(
                wt_hbm.at[:, pl.ds(jnp.minimum(blk, TAIL_BLK - 1) * 128,
                                   128)],
                slab_v.at[s], sems[s])
            return 0

        lax.cond(blk == TAIL_BLK, fetch_tail, fetch_slab, 0)

    for s in range(RING):
        @pl.when(s < bn)
        def _():
            issue(s, s)

    def process(i, s, k):
        pltpu.make_async_copy(tail_hbm, slab_v.at[s], sems[s]).wait()
        jrel = blist_v[pl.ds(i, L)][0]
        blk = lo + jrel
        col_base = jnp.where(blk == TAIL_BLK, TAIL_START, blk * 128)
        m0 = start_v[pl.ds(jrel, L)][0]
        c = cnt_v[pl.ds(jrel, L)][0]

        def per_match(m2, k2):
            b = ml_v[pl.ds(m2, L)][0]
            r = idx_v[pl.ds(b, L)][0]
            col = r - col_base
            kk = k2 & (FLUSH - 1)
            for q in range(EMB // L):
                vals = plsc.load_gather(
                    slab_v.at[s], [q * L + lanes,
                                   jnp.broadcast_to(col, (L,))])
                rows_v[kk, pl.ds(q * L, L)] = vals
            plsc.store_scatter(mb_v, [jnp.broadcast_to(kk, (L,))],
                               jnp.broadcast_to(b, (L,)), mask=lanes == 0)
            k2 = k2 + 1

            def flush(_):
                pltpu.async_copy(rows_v, g_hbm.at[mb_v], sem_sc).wait()
                for q2 in range(FLUSH // L):
                    mb_v[pl.ds(q2 * L, L)] = jnp.full((L,), DUMP, jnp.int32)
                return 0

            lax.cond((k2 & (FLUSH - 1)) == 0, flush, lambda _: 0, 0)
            return k2

        k = lax.fori_loop(m0, m0 + c, per_match, k)

        @pl.when(i + RING < bn)
        def _():
            issue(i + RING, s)

        return k

    def ring_group(g, k):
        for s in range(RING):
            i = g * RING + s

            def do(kk):
                return process(i, s, kk)

            k = lax.cond(i < bn, do, lambda kk: kk, k)
        return k

    lax.fori_loop(0, (bn + RING - 1) // RING, ring_group, 0)
    pltpu.async_copy(rows_v, g_hbm.at[mb_v], sem_sc).wait()


def _combine_body(gw_hbm, gc_hbm, out_hbm, rw_v, rc_v, out_v, sem_w, sem_c):
    wid = lax.axis_index("s") * NC + lax.axis_index("c")
    lanes = lax.iota(jnp.int32, L)
    bpw = BATCH // NW
    ch = FLUSH

    for h in range(bpw // ch):
        base = wid * bpw + h * ch
        cp_w = pltpu.async_copy(gw_hbm.at[pl.ds(base, ch)], rw_v, sem_w)
        cp_c = pltpu.async_copy(gc_hbm.at[pl.ds(base, ch)], rc_v, sem_c)
        cp_w.wait()
        cp_c.wait()

        def group(g, carry):
            row = g * L + lanes
            acc = jnp.zeros((L,), jnp.float32)
            for e in range(EMB):
                col = jnp.full((L,), e, jnp.int32)
                w = plsc.load_gather(rw_v, [row, col])
                c = plsc.load_gather(rc_v, [row, col])
                acc = acc + w * c
            out_v[pl.ds(g * L, L)] = _neg_log_sigmoid(acc)
            return carry

        lax.fori_loop(0, ch // L, group, 0)
        pltpu.sync_copy(out_v, out_hbm.at[pl.ds(base, ch)])


_SC_PARAMS = pltpu.CompilerParams(needs_layout_passes=False,
                                  use_tc_tiling_on_sc=True)

_gather_call = functools.partial(
    pl.kernel,
    mesh=plsc.VectorSubcoreMesh(core_axis_name="c", subcore_axis_name="s"),
    out_type=jax.ShapeDtypeStruct((GOUT, 128), jnp.float32),
    scratch_types=[
        pltpu.VMEM((BATCH + L,), jnp.int32),
        pltpu.VMEM((BATCH + L,), jnp.int32),
        pltpu.VMEM((NB + L,), jnp.int32),
        pltpu.VMEM((NB + L,), jnp.int32),
        pltpu.VMEM((NB + L,), jnp.int32),
        pltpu.VMEM((NB + L,), jnp.int32),
        pltpu.VMEM((2 * L,), jnp.int32),
        pltpu.VMEM((RING, EMB, 128), jnp.float32),
        pltpu.VMEM((FLUSH, 128), jnp.float32),
        pltpu.VMEM((FLUSH,), jnp.int32),
        pltpu.SemaphoreType.DMA,
        pltpu.SemaphoreType.DMA,
        pltpu.SemaphoreType.DMA,
        pltpu.SemaphoreType.DMA,
        pltpu.SemaphoreType.DMA,
    ],
    compiler_params=_SC_PARAMS,
)(_gather_body)

_combine_call = functools.partial(
    pl.kernel,
    mesh=plsc.VectorSubcoreMesh(core_axis_name="c", subcore_axis_name="s"),
    out_type=jax.ShapeDtypeStruct((BATCH,), jnp.float32),
    scratch_types=[
        pltpu.VMEM((FLUSH, 128), jnp.float32),
        pltpu.VMEM((FLUSH, 128), jnp.float32),
        pltpu.VMEM((FLUSH,), jnp.float32),
        pltpu.SemaphoreType.DMA,
        pltpu.SemaphoreType.DMA,
    ],
    compiler_params=_SC_PARAMS,
)(_combine_body)


def _tail(W):
    return jnp.pad(W[TAIL_START:].T, ((0, 0), (0, 128 - (NWORDS - TAIL_START))))


def kernel(word_pos, context_pos, W_word, W_context):
    wp = word_pos.astype(jnp.int32)
    cp = context_pos.astype(jnp.int32)
    gw = _gather_call(W_word.T, _tail(W_word), wp)
    gc = _gather_call(W_context.T, _tail(W_context), cp)
    return _combine_call(gw, gc)


# speedup vs baseline: 2.2325x; 1.3750x over previous
import functools

import jax
import jax.numpy as jnp
from jax import lax
from jax.experimental import pallas as pl
from jax.experimental.pallas import tpu as pltpu
from jax.experimental.pallas import tpu_sc as plsc

NWORDS = 1000000
EMB = 64
BATCH = 16384

NC = 2
NS = 16
L = 16
NW = NC * NS

NBLK = (NWORDS + 127) // 128
NB = (NBLK + NW - 1) // NW
TAIL_BLK = NWORDS // 128
TAIL_START = TAIL_BLK * 128
GOUT = BATCH + 8
DUMP = GOUT - 1
FLUSH = 128
NVREG = BATCH // L


def _neg_log_sigmoid(s):
    z = -s
    m = jnp.maximum(z, 0.0)
    u = jnp.exp(-jnp.abs(z))
    t = u / (2.0 + u)
    t2 = t * t
    p = 1.0 + t2 * (1.0 / 3.0 + t2 * (1.0 / 5.0 + t2 * (1.0 / 7.0 + t2 * (1.0 / 9.0))))
    return m + 2.0 * t * p


RING = 8


def _gather_body(wt_hbm, tail_hbm, idx_hbm, g_hbm,
                 idx_v, ml_v, cnt_v, start_v, fill_v, blist_v, stage_v,
                 slab_v, rows_v, mb_v, sem0, sem1, sem2, sem3,
                 sem4, sem5, sem6, sem7, sem_sc):
    sems = [sem0, sem1, sem2, sem3, sem4, sem5, sem6, sem7]
    wid = lax.axis_index("s") * NC + lax.axis_index("c")
    lo = wid * NB
    lanes = lax.iota(jnp.int32, L)

    pltpu.sync_copy(idx_hbm, idx_v.at[pl.ds(0, BATCH)])
    for q in range((NB + L - 1) // L):
        cnt_v[pl.ds(q * L, L)] = jnp.zeros((L,), jnp.int32)
    for q in range(FLUSH // L):
        mb_v[pl.ds(q * L, L)] = jnp.full((L,), DUMP, jnp.int32)

    def scan1(v, _):
        iv = idx_v[pl.ds(v * L, L)]
        bv = (iv >> 7) - lo
        m = (bv >= 0) & (bv < NB)
        plsc.addupdate_scatter(cnt_v, [jnp.where(m, bv, 0)],
                               jnp.ones((L,), jnp.int32), mask=m)
        return 0

    lax.fori_loop(0, NVREG, scan1, 0)

    carry = 0
    bn = 0
    for q in range((NB + L - 1) // L):
        cv = cnt_v[pl.ds(q * L, L)]
        cs = plsc.cumsum(cv)
        excl = carry + cs - cv
        start_v[pl.ds(q * L, L)] = excl
        fill_v[pl.ds(q * L, L)] = excl
        carry = carry + cs[L - 1]
        occ = (cv > 0) & (q * L + lanes < NB)
        plsc.store_compressed(blist_v.at[pl.ds(bn, L)], q * L + lanes,
                              mask=occ)
        pcq = plsc.all_reduce_population_count(occ)
        bn = bn + pcq[0]

    def scan2(v, _):
        iv = idx_v[pl.ds(v * L, L)]
        bv = (iv >> 7) - lo
        m = (bv >= 0) & (bv < NB)
        plsc.store_compressed(stage_v.at[pl.ds(0, L)], v * L + lanes,
                              mask=m)
        pc = plsc.all_reduce_population_count(m)

        def put(t, _2):
            b = stage_v[pl.ds(t, L)][0]
            jb = (idx_v[pl.ds(b, L)][0] >> 7) - lo
            p = fill_v[pl.ds(jb, L)][0]
            plsc.store_scatter(ml_v, [jnp.broadcast_to(p, (L,))],
                               jnp.broadcast_to(b, (L,)), mask=lanes == 0)
            plsc.store_scatter(fill_v, [jnp.broadcast_to(jb, (L,))],
                               jnp.broadcast_to(p + 1, (L,)),
                               mask=lanes == 0)
            return 0

        lax.fori_loop(0, pc[0], put, 0)
        return 0

    lax.fori_loop(0, NVREG, scan2, 0)

    def issue(i, s):
        jrel = blist_v[pl.ds(i, L)][0]
        blk = lo + jrel

        def fetch_tail(_):
            pltpu.async_copy(tail_hbm, slab_v.at[s], sems[s])
            return 0

        def fetch_slab(_):
            pltpu.async_copy(
                wt_hbm.at[:, pl.ds(jnp.minimum(blk, TAIL_BLK - 1) * 128,
                                   128)],
                slab_v.at[s], sems[s])
            return 0

        lax.cond(blk == TAIL_BLK, fetch_tail, fetch_slab, 0)

    for s in range(RING):
        @pl.when(s < bn)
        def _():
            issue(s, s)

    def process(i, s, k):
        pltpu.make_async_copy(tail_hbm, slab_v.at[s], sems[s]).wait()
        jrel = blist_v[pl.ds(i, L)][0]
        blk = lo + jrel
        col_base = jnp.where(blk == TAIL_BLK, TAIL_START, blk * 128)
        m0 = start_v[pl.ds(jrel, L)][0]
        c = cnt_v[pl.ds(jrel, L)][0]

        def per_match(m2, k2):
            b = ml_v[pl.ds(m2, L)][0]
            r = idx_v[pl.ds(b, L)][0]
            col = r - col_base
            kk = k2 & (FLUSH - 1)
            for q in range(EMB // L):
                vals = plsc.load_gather(
                    slab_v.at[s], [q * L + lanes,
                                   jnp.broadcast_to(col, (L,))])
                rows_v[kk, pl.ds(q * L, L)] = vals
            plsc.store_scatter(mb_v, [jnp.broadcast_to(kk, (L,))],
                               jnp.broadcast_to(b, (L,)), mask=lanes == 0)
            k2 = k2 + 1

            def flush(_):
                pltpu.async_copy(rows_v, g_hbm.at[mb_v], sem_sc).wait()
                for q2 in range(FLUSH // L):
                    mb_v[pl.ds(q2 * L, L)] = jnp.full((L,), DUMP, jnp.int32)
                return 0

            lax.cond((k2 & (FLUSH - 1)) == 0, flush, lambda _: 0, 0)
            return k2

        k = lax.fori_loop(m0, m0 + c, per_match, k)

        @pl.when(i + RING < bn)
        def _():
            issue(i + RING, s)

        return k

    def ring_group(g, k):
        for s in range(RING):
            i = g * RING + s

            def do(kk):
                return process(i, s, kk)

            k = lax.cond(i < bn, do, lambda kk: kk, k)
        return k

    lax.fori_loop(0, (bn + RING - 1) // RING, ring_group, 0)
    pltpu.async_copy(rows_v, g_hbm.at[mb_v], sem_sc).wait()


def _combine_body(gw_hbm, gc_hbm, out_hbm, rw_v, rc_v, out_v, sem_w, sem_c):
    wid = lax.axis_index("s") * NC + lax.axis_index("c")
    lanes = lax.iota(jnp.int32, L)
    bpw = BATCH // NW
    ch = FLUSH

    for h in range(bpw // ch):
        base = wid * bpw + h * ch
        cp_w = pltpu.async_copy(gw_hbm.at[pl.ds(base, ch)], rw_v, sem_w)
        cp_c = pltpu.async_copy(gc_hbm.at[pl.ds(base, ch)], rc_v, sem_c)
        cp_w.wait()
        cp_c.wait()

        def group(g, carry):
            row = g * L + lanes
            acc = jnp.zeros((L,), jnp.float32)
            for e in range(EMB):
                col = jnp.full((L,), e, jnp.int32)
                w = plsc.load_gather(rw_v, [row, col])
                c = plsc.load_gather(rc_v, [row, col])
                acc = acc + w * c
            out_v[pl.ds(g * L, L)] = _neg_log_sigmoid(acc)
            return carry

        lax.fori_loop(0, ch // L, group, 0)
        pltpu.sync_copy(out_v, out_hbm.at[pl.ds(base, ch)])


_SC_PARAMS = pltpu.CompilerParams(needs_layout_passes=False,
                                  use_tc_tiling_on_sc=True)

_gather_call = functools.partial(
    pl.kernel,
    mesh=plsc.VectorSubcoreMesh(core_axis_name="c", subcore_axis_name="s"),
    out_type=jax.ShapeDtypeStruct((GOUT, 128), jnp.float32),
    scratch_types=[
        pltpu.VMEM((BATCH + L,), jnp.int32),
        pltpu.VMEM((BATCH + L,), jnp.int32),
        pltpu.VMEM((NB + L,), jnp.int32),
        pltpu.VMEM((NB + L,), jnp.int32),
        pltpu.VMEM((NB + L,), jnp.int32),
        pltpu.VMEM((NB + L,), jnp.int32),
        pltpu.VMEM((2 * L,), jnp.int32),
        pltpu.VMEM((RING, EMB, 128), jnp.float32),
        pltpu.VMEM((FLUSH, 128), jnp.float32),
        pltpu.VMEM((FLUSH,), jnp.int32),
        pltpu.SemaphoreType.DMA,
        pltpu.SemaphoreType.DMA,
        pltpu.SemaphoreType.DMA,
        pltpu.SemaphoreType.DMA,
        pltpu.SemaphoreType.DMA,
        pltpu.SemaphoreType.DMA,
        pltpu.SemaphoreType.DMA,
        pltpu.SemaphoreType.DMA,
        pltpu.SemaphoreType.DMA,
    ],
    compiler_params=_SC_PARAMS,
)(_gather_body)

_combine_call = functools.partial(
    pl.kernel,
    mesh=plsc.VectorSubcoreMesh(core_axis_name="c", subcore_axis_name="s"),
    out_type=jax.ShapeDtypeStruct((BATCH,), jnp.float32),
    scratch_types=[
        pltpu.VMEM((FLUSH, 128), jnp.float32),
        pltpu.VMEM((FLUSH, 128), jnp.float32),
        pltpu.VMEM((FLUSH,), jnp.float32),
        pltpu.SemaphoreType.DMA,
        pltpu.SemaphoreType.DMA,
    ],
    compiler_params=_SC_PARAMS,
)(_combine_body)


def _tail(W):
    return jnp.pad(W[TAIL_START:].T, ((0, 0), (0, 128 - (NWORDS - TAIL_START))))


def kernel(word_pos, context_pos, W_word, W_context):
    wp = word_pos.astype(jnp.int32)
    cp = context_pos.astype(jnp.int32)
    gw = _gather_call(W_word.T, _tail(W_word), wp)
    gc = _gather_call(W_context.T, _tail(W_context), cp)
    return _combine_call(gw, gc)


# speedup vs baseline: 2.6175x; 1.1725x over previous
import functools

import jax
import jax.numpy as jnp
from jax import lax
from jax.experimental import pallas as pl
from jax.experimental.pallas import tpu as pltpu
from jax.experimental.pallas import tpu_sc as plsc

NWORDS = 1000000
EMB = 64
BATCH = 16384

NC = 2
NS = 16
L = 16
NW = NC * NS

NBLK = (NWORDS + 127) // 128
NB = (NBLK + NW - 1) // NW
TAIL_BLK = NWORDS // 128
TAIL_START = TAIL_BLK * 128
GOUT = BATCH + 8
DUMP = GOUT - 1
FLUSH = 64
NVREG = BATCH // L


def _neg_log_sigmoid(s):
    z = -s
    m = jnp.maximum(z, 0.0)
    u = jnp.exp(-jnp.abs(z))
    t = u / (2.0 + u)
    t2 = t * t
    p = 1.0 + t2 * (1.0 / 3.0 + t2 * (1.0 / 5.0 + t2 * (1.0 / 7.0 + t2 * (1.0 / 9.0))))
    return m + 2.0 * t * p


RING = 10


def _gather_body(wt_hbm, tail_hbm, idx_hbm, g_hbm,
                 idx_v, ml_v, cnt_v, start_v, fill_v, blist_v, stage_v,
                 slab_v, rows_v, mb_v, sem0, sem1, sem2, sem3,
                 sem4, sem5, sem6, sem7, sem8, sem9, sem_sc):
    sems = [sem0, sem1, sem2, sem3, sem4, sem5, sem6, sem7, sem8, sem9]
    wid = lax.axis_index("s") * NC + lax.axis_index("c")
    lo = wid * NB
    lanes = lax.iota(jnp.int32, L)

    pltpu.sync_copy(idx_hbm, idx_v.at[pl.ds(0, BATCH)])
    for q in range((NB + L - 1) // L):
        cnt_v[pl.ds(q * L, L)] = jnp.zeros((L,), jnp.int32)
    for q in range(FLUSH // L):
        mb_v[pl.ds(q * L, L)] = jnp.full((L,), DUMP, jnp.int32)

    def scan1(v, _):
        iv = idx_v[pl.ds(v * L, L)]
        bv = (iv >> 7) - lo
        m = (bv >= 0) & (bv < NB)
        plsc.addupdate_scatter(cnt_v, [jnp.where(m, bv, 0)],
                               jnp.ones((L,), jnp.int32), mask=m)
        return 0

    lax.fori_loop(0, NVREG, scan1, 0)

    carry = 0
    bn = 0
    for q in range((NB + L - 1) // L):
        cv = cnt_v[pl.ds(q * L, L)]
        cs = plsc.cumsum(cv)
        excl = carry + cs - cv
        start_v[pl.ds(q * L, L)] = excl
        fill_v[pl.ds(q * L, L)] = excl
        carry = carry + cs[L - 1]
        occ = (cv > 0) & (q * L + lanes < NB)
        plsc.store_compressed(blist_v.at[pl.ds(bn, L)], q * L + lanes,
                              mask=occ)
        pcq = plsc.all_reduce_population_count(occ)
        bn = bn + pcq[0]

    def scan2(v, _):
        iv = idx_v[pl.ds(v * L, L)]
        bv = (iv >> 7) - lo
        m = (bv >= 0) & (bv < NB)
        plsc.store_compressed(stage_v.at[pl.ds(0, L)], v * L + lanes,
                              mask=m)
        pc = plsc.all_reduce_population_count(m)

        def put(t, _2):
            b = stage_v[pl.ds(t, L)][0]
            jb = (idx_v[pl.ds(b, L)][0] >> 7) - lo
            p = fill_v[pl.ds(jb, L)][0]
            plsc.store_scatter(ml_v, [jnp.broadcast_to(p, (L,))],
                               jnp.broadcast_to(b, (L,)), mask=lanes == 0)
            plsc.store_scatter(fill_v, [jnp.broadcast_to(jb, (L,))],
                               jnp.broadcast_to(p + 1, (L,)),
                               mask=lanes == 0)
            return 0

        lax.fori_loop(0, pc[0], put, 0)
        return 0

    lax.fori_loop(0, NVREG, scan2, 0)

    def issue(i, s):
        jrel = blist_v[pl.ds(i, L)][0]
        blk = lo + jrel

        def fetch_tail(_):
            pltpu.async_copy(tail_hbm, slab_v.at[s], sems[s])
            return 0

        def fetch_slab(_):
            pltpu.async_copy(
                wt_hbm.at[:, pl.ds(jnp.minimum(blk, TAIL_BLK - 1) * 128,
                                   128)],
                slab_v.at[s], sems[s])
            return 0

        lax.cond(blk == TAIL_BLK, fetch_tail, fetch_slab, 0)

    for s in range(RING):
        @pl.when(s < bn)
        def _():
            issue(s, s)

    def process(i, s, k):
        pltpu.make_async_copy(tail_hbm, slab_v.at[s], sems[s]).wait()
        jrel = blist_v[pl.ds(i, L)][0]
        blk = lo + jrel
        col_base = jnp.where(blk == TAIL_BLK, TAIL_START, blk * 128)
        m0 = start_v[pl.ds(jrel, L)][0]
        c = cnt_v[pl.ds(jrel, L)][0]

        def per_match(m2, k2):
            b = ml_v[pl.ds(m2, L)][0]
            r = idx_v[pl.ds(b, L)][0]
            col = r - col_base
            kk = k2 & (FLUSH - 1)
            for q in range(EMB // L):
                vals = plsc.load_gather(
                    slab_v.at[s], [q * L + lanes,
                                   jnp.broadcast_to(col, (L,))])
                rows_v[kk, pl.ds(q * L, L)] = vals
            plsc.store_scatter(mb_v, [jnp.broadcast_to(kk, (L,))],
                               jnp.broadcast_to(b, (L,)), mask=lanes == 0)
            k2 = k2 + 1

            def flush(_):
                pltpu.async_copy(rows_v, g_hbm.at[mb_v], sem_sc).wait()
                for q2 in range(FLUSH // L):
                    mb_v[pl.ds(q2 * L, L)] = jnp.full((L,), DUMP, jnp.int32)
                return 0

            lax.cond((k2 & (FLUSH - 1)) == 0, flush, lambda _: 0, 0)
            return k2

        k = lax.fori_loop(m0, m0 + c, per_match, k)

        @pl.when(i + RING < bn)
        def _():
            issue(i + RING, s)

        return k

    def ring_group(g, k):
        for s in range(RING):
            i = g * RING + s

            def do(kk):
                return process(i, s, kk)

            k = lax.cond(i < bn, do, lambda kk: kk, k)
        return k

    lax.fori_loop(0, (bn + RING - 1) // RING, ring_group, 0)
    pltpu.async_copy(rows_v, g_hbm.at[mb_v], sem_sc).wait()


def _combine_body(gw_hbm, gc_hbm, out_hbm, rw_v, rc_v, out_v, sem_w, sem_c):
    wid = lax.axis_index("s") * NC + lax.axis_index("c")
    lanes = lax.iota(jnp.int32, L)
    bpw = BATCH // NW
    ch = FLUSH

    for h in range(bpw // ch):
        base = wid * bpw + h * ch
        cp_w = pltpu.async_copy(gw_hbm.at[pl.ds(base, ch)], rw_v, sem_w)
        cp_c = pltpu.async_copy(gc_hbm.at[pl.ds(base, ch)], rc_v, sem_c)
        cp_w.wait()
        cp_c.wait()

        def group(g, carry):
            row = g * L + lanes
            acc = jnp.zeros((L,), jnp.float32)
            for e in range(EMB):
                col = jnp.full((L,), e, jnp.int32)
                w = plsc.load_gather(rw_v, [row, col])
                c = plsc.load_gather(rc_v, [row, col])
                acc = acc + w * c
            out_v[pl.ds(g * L, L)] = _neg_log_sigmoid(acc)
            return carry

        lax.fori_loop(0, ch // L, group, 0)
        pltpu.sync_copy(out_v, out_hbm.at[pl.ds(base, ch)])


_SC_PARAMS = pltpu.CompilerParams(needs_layout_passes=False,
                                  use_tc_tiling_on_sc=True)

_gather_call = functools.partial(
    pl.kernel,
    mesh=plsc.VectorSubcoreMesh(core_axis_name="c", subcore_axis_name="s"),
    out_type=jax.ShapeDtypeStruct((GOUT, 128), jnp.float32),
    scratch_types=[
        pltpu.VMEM((BATCH + L,), jnp.int32),
        pltpu.VMEM((BATCH + L,), jnp.int32),
        pltpu.VMEM((NB + L,), jnp.int32),
        pltpu.VMEM((NB + L,), jnp.int32),
        pltpu.VMEM((NB + L,), jnp.int32),
        pltpu.VMEM((NB + L,), jnp.int32),
        pltpu.VMEM((2 * L,), jnp.int32),
        pltpu.VMEM((RING, EMB, 128), jnp.float32),
        pltpu.VMEM((FLUSH, 128), jnp.float32),
        pltpu.VMEM((FLUSH,), jnp.int32),
    ] + [pltpu.SemaphoreType.DMA] * (RING + 1),
    compiler_params=_SC_PARAMS,
)(_gather_body)

_combine_call = functools.partial(
    pl.kernel,
    mesh=plsc.VectorSubcoreMesh(core_axis_name="c", subcore_axis_name="s"),
    out_type=jax.ShapeDtypeStruct((BATCH,), jnp.float32),
    scratch_types=[
        pltpu.VMEM((FLUSH, 128), jnp.float32),
        pltpu.VMEM((FLUSH, 128), jnp.float32),
        pltpu.VMEM((FLUSH,), jnp.float32),
        pltpu.SemaphoreType.DMA,
        pltpu.SemaphoreType.DMA,
    ],
    compiler_params=_SC_PARAMS,
)(_combine_body)


def _tail(W):
    return jnp.pad(W[TAIL_START:].T, ((0, 0), (0, 128 - (NWORDS - TAIL_START))))


def kernel(word_pos, context_pos, W_word, W_context):
    wp = word_pos.astype(jnp.int32)
    cp = context_pos.astype(jnp.int32)
    gw = _gather_call(W_word.T, _tail(W_word), wp)
    gc = _gather_call(W_context.T, _tail(W_context), cp)
    return _combine_call(gw, gc)


# speedup vs baseline: 2.6813x; 1.0244x over previous
import functools

import jax
import jax.numpy as jnp
from jax import lax
from jax.experimental import pallas as pl
from jax.experimental.pallas import tpu as pltpu
from jax.experimental.pallas import tpu_sc as plsc

NWORDS = 1000000
EMB = 64
BATCH = 16384

NC = 2
NS = 16
L = 16
NW = NC * NS

NBLK = (NWORDS + 127) // 128
NB = (NBLK + NW - 1) // NW
TAIL_BLK = NWORDS // 128
TAIL_START = TAIL_BLK * 128
GOUT = BATCH + 8
DUMP = GOUT - 1
FLUSH = 64
NVREG = BATCH // L


def _neg_log_sigmoid(s):
    z = -s
    m = jnp.maximum(z, 0.0)
    u = jnp.exp(-jnp.abs(z))
    t = u / (2.0 + u)
    t2 = t * t
    p = 1.0 + t2 * (1.0 / 3.0 + t2 * (1.0 / 5.0 + t2 * (1.0 / 7.0 + t2 * (1.0 / 9.0))))
    return m + 2.0 * t * p


RING = 10


def _gather_body(wt_hbm, tail_hbm, idx_hbm, g_hbm,
                 idx_v, ml_v, cnt_v, start_v, fill_v, blist_v, stage_v,
                 slab_v, rows_v, mb_v, sem0, sem1, sem2, sem3,
                 sem4, sem5, sem6, sem7, sem8, sem9, sem_sc):
    sems = [sem0, sem1, sem2, sem3, sem4, sem5, sem6, sem7, sem8, sem9]
    wid = lax.axis_index("s") * NC + lax.axis_index("c")
    lo = wid * NB
    lanes = lax.iota(jnp.int32, L)

    pltpu.sync_copy(idx_hbm, idx_v.at[pl.ds(0, BATCH)])
    for q in range((NB + L - 1) // L):
        cnt_v[pl.ds(q * L, L)] = jnp.zeros((L,), jnp.int32)
    for q in range(FLUSH // L):
        mb_v[pl.ds(q * L, L)] = jnp.full((L,), DUMP, jnp.int32)

    def scan1(v, _):
        iv = idx_v[pl.ds(v * L, L)]
        bv = (iv >> 7) - lo
        m = (bv >= 0) & (bv < NB)
        plsc.addupdate_scatter(cnt_v, [jnp.where(m, bv, 0)],
                               jnp.ones((L,), jnp.int32), mask=m)
        return 0

    lax.fori_loop(0, NVREG, scan1, 0)

    carry = 0
    bn = 0
    for q in range((NB + L - 1) // L):
        cv = cnt_v[pl.ds(q * L, L)]
        cs = plsc.cumsum(cv)
        excl = carry + cs - cv
        start_v[pl.ds(q * L, L)] = excl
        fill_v[pl.ds(q * L, L)] = excl
        carry = carry + cs[L - 1]
        occ = (cv > 0) & (q * L + lanes < NB)
        plsc.store_compressed(blist_v.at[pl.ds(bn, L)], q * L + lanes,
                              mask=occ)
        pcq = plsc.all_reduce_population_count(occ)
        bn = bn + pcq[0]

    def scan2(v, _):
        iv = idx_v[pl.ds(v * L, L)]
        bv = (iv >> 7) - lo
        m = (bv >= 0) & (bv < NB)
        plsc.store_compressed(stage_v.at[pl.ds(0, L)], v * L + lanes,
                              mask=m)
        pc = plsc.all_reduce_population_count(m)

        def put(t, _2):
            b = stage_v[pl.ds(t, L)][0]
            jb = (idx_v[pl.ds(b, L)][0] >> 7) - lo
            p = fill_v[pl.ds(jb, L)][0]
            plsc.store_scatter(ml_v, [jnp.broadcast_to(p, (L,))],
                               jnp.broadcast_to(b, (L,)), mask=lanes == 0)
            plsc.store_scatter(fill_v, [jnp.broadcast_to(jb, (L,))],
                               jnp.broadcast_to(p + 1, (L,)),
                               mask=lanes == 0)
            return 0

        lax.fori_loop(0, pc[0], put, 0)
        return 0

    lax.fori_loop(0, NVREG, scan2, 0)

    def issue(i, s):
        jrel = blist_v[pl.ds(i, L)][0]
        blk = lo + jrel

        def fetch_tail(_):
            pltpu.async_copy(tail_hbm, slab_v.at[s], sems[s])
            return 0

        def fetch_slab(_):
            pltpu.async_copy(
                wt_hbm.at[:, pl.ds(jnp.minimum(blk, TAIL_BLK - 1) * 128,
                                   128)],
                slab_v.at[s], sems[s])
            return 0

        lax.cond(blk == TAIL_BLK, fetch_tail, fetch_slab, 0)

    for s in range(RING):
        @pl.when(s < bn)
        def _():
            issue(s, s)

    def process(i, s, k):
        pltpu.make_async_copy(tail_hbm, slab_v.at[s], sems[s]).wait()
        jrel = blist_v[pl.ds(i, L)][0]
        blk = lo + jrel
        col_base = jnp.where(blk == TAIL_BLK, TAIL_START, blk * 128)
        m0 = start_v[pl.ds(jrel, L)][0]
        c = cnt_v[pl.ds(jrel, L)][0]

        def per_match(m2, k2):
            b = ml_v[pl.ds(m2, L)][0]
            r = idx_v[pl.ds(b, L)][0]
            col = r - col_base
            kk = k2 & (FLUSH - 1)
            for q in range(EMB // L):
                vals = plsc.load_gather(
                    slab_v.at[s], [q * L + lanes,
                                   jnp.broadcast_to(col, (L,))])
                rows_v[kk, pl.ds(q * L, L)] = vals
            plsc.store_scatter(mb_v, [jnp.broadcast_to(kk, (L,))],
                               jnp.broadcast_to(b, (L,)), mask=lanes == 0)
            k2 = k2 + 1

            def flush(_):
                pltpu.async_copy(rows_v, g_hbm.at[mb_v], sem_sc).wait()
                for q2 in range(FLUSH // L):
                    mb_v[pl.ds(q2 * L, L)] = jnp.full((L,), DUMP, jnp.int32)
                return 0

            lax.cond((k2 & (FLUSH - 1)) == 0, flush, lambda _: 0, 0)
            return k2

        k = lax.fori_loop(m0, m0 + c, per_match, k)

        @pl.when(i + RING < bn)
        def _():
            issue(i + RING, s)

        return k

    def ring_group(g, k):
        for s in range(RING):
            i = g * RING + s

            def do(kk):
                return process(i, s, kk)

            k = lax.cond(i < bn, do, lambda kk: kk, k)
        return k

    lax.fori_loop(0, (bn + RING - 1) // RING, ring_group, 0)
    pltpu.async_copy(rows_v, g_hbm.at[mb_v], sem_sc).wait()


def _combine_body(gw_hbm, gc_hbm, out_hbm, rw_v, rc_v, out_v, sem_w, sem_c):
    wid = lax.axis_index("s") * NC + lax.axis_index("c")
    lanes = lax.iota(jnp.int32, L)
    bpw = BATCH // NW
    ch = 256

    for h in range(bpw // ch):
        base = wid * bpw + h * ch
        cp_w = pltpu.async_copy(gw_hbm.at[pl.ds(base, ch)], rw_v, sem_w)
        cp_c = pltpu.async_copy(gc_hbm.at[pl.ds(base, ch)], rc_v, sem_c)
        cp_w.wait()
        cp_c.wait()

        def group(g, carry):
            row = g * L + lanes
            acc = jnp.zeros((L,), jnp.float32)
            for e in range(EMB):
                col = jnp.full((L,), e, jnp.int32)
                w = plsc.load_gather(rw_v, [row, col])
                c = plsc.load_gather(rc_v, [row, col])
                acc = acc + w * c
            out_v[pl.ds(g * L, L)] = _neg_log_sigmoid(acc)
            return carry

        lax.fori_loop(0, ch // L, group, 0)
        pltpu.sync_copy(out_v, out_hbm.at[pl.ds(base, ch)])


_SC_PARAMS = pltpu.CompilerParams(needs_layout_passes=False,
                                  use_tc_tiling_on_sc=True)

_gather_call = functools.partial(
    pl.kernel,
    mesh=plsc.VectorSubcoreMesh(core_axis_name="c", subcore_axis_name="s"),
    out_type=jax.ShapeDtypeStruct((GOUT, 128), jnp.float32),
    scratch_types=[
        pltpu.VMEM((BATCH + L,), jnp.int32),
        pltpu.VMEM((BATCH + L,), jnp.int32),
        pltpu.VMEM((NB + L,), jnp.int32),
        pltpu.VMEM((NB + L,), jnp.int32),
        pltpu.VMEM((NB + L,), jnp.int32),
        pltpu.VMEM((NB + L,), jnp.int32),
        pltpu.VMEM((2 * L,), jnp.int32),
        pltpu.VMEM((RING, EMB, 128), jnp.float32),
        pltpu.VMEM((FLUSH, 128), jnp.float32),
        pltpu.VMEM((FLUSH,), jnp.int32),
    ] + [pltpu.SemaphoreType.DMA] * (RING + 1),
    compiler_params=_SC_PARAMS,
)(_gather_body)

_combine_call = functools.partial(
    pl.kernel,
    mesh=plsc.VectorSubcoreMesh(core_axis_name="c", subcore_axis_name="s"),
    out_type=jax.ShapeDtypeStruct((BATCH,), jnp.float32),
    scratch_types=[
        pltpu.VMEM((256, 128), jnp.float32),
        pltpu.VMEM((256, 128), jnp.float32),
        pltpu.VMEM((256,), jnp.float32),
        pltpu.SemaphoreType.DMA,
        pltpu.SemaphoreType.DMA,
    ],
    compiler_params=_SC_PARAMS,
)(_combine_body)


def _tail(W):
    return jnp.pad(W[TAIL_START:].T, ((0, 0), (0, 128 - (NWORDS - TAIL_START))))


def kernel(word_pos, context_pos, W_word, W_context):
    wp = word_pos.astype(jnp.int32)
    cp = context_pos.astype(jnp.int32)
    gw = _gather_call(W_word.T, _tail(W_word), wp)
    gc = _gather_call(W_context.T, _tail(W_context), cp)
    return _combine_call(gw, gc)
